# Initial kernel scaffold; baseline (speedup 1.0000x reference)
import jax
import jax.numpy as jnp
from jax.experimental import pallas as pl


def kernel(feats, mask, x_len, W, b, gamma, beta):
    raise NotImplementedError("write your pallas kernel here")


# speedup vs baseline: 4.1433x; 4.1433x over previous
import functools

import jax
import jax.numpy as jnp
from jax.experimental import pallas as pl
from jax.experimental.pallas import tpu as pltpu

_BATCH = 16
_ROWS_PER_BLOCK = 1024


def _fused_kernel(lens_ref, feats_ref, mask_ref, w1_ref, w2_ref, b_ref,
                  gamma_ref, beta_ref, xraw_ref, x_ref, acc_ref,
                  *, rows_per_block, num_blocks):
    i = pl.program_id(0)

    y = jnp.dot(feats_ref[...], w1_ref[...], preferred_element_type=jnp.float32)
    y = y + jnp.dot(mask_ref[...], w2_ref[...], preferred_element_type=jnp.float32)
    y = y + b_ref[...]
    y = jnp.where(y >= 0.0, y, 0.01 * y)
    xraw_ref[...] = y

    lens_col = lens_ref[:, 0:1].astype(jnp.float32)
    r = jax.lax.broadcasted_iota(jnp.int32, (_BATCH, _BATCH), 0)
    c = jax.lax.broadcasted_iota(jnp.int32, (_BATCH, _BATCH), 1)
    tril = (c < r).astype(jnp.float32)
    starts = jnp.dot(tril, lens_col, preferred_element_type=jnp.float32,
                     precision=jax.lax.Precision.HIGHEST)
    ends = starts + lens_col

    row_idx = (i * rows_per_block
               + jax.lax.broadcasted_iota(jnp.int32, (_BATCH, rows_per_block), 1)
               ).astype(jnp.float32)
    onehot = ((row_idx >= starts) & (row_idx < ends)).astype(jnp.float32)
    partial = jnp.dot(onehot, y, preferred_element_type=jnp.float32,
                      precision=jax.lax.Precision.HIGHEST)

    @pl.when(i == 0)
    def _init():
        acc_ref[...] = partial

    @pl.when(i > 0)
    def _accum():
        acc_ref[...] = acc_ref[...] + partial

    @pl.when(i == num_blocks - 1)
    def _finalize():
        mean = acc_ref[...] / lens_col
        mu = jnp.mean(mean, axis=-1, keepdims=True)
        var = jnp.mean((mean - mu) ** 2, axis=-1, keepdims=True)
        x_ref[...] = ((mean - mu) / jnp.sqrt(var + 1e-5)
                      * gamma_ref[...] + beta_ref[...])


def kernel(feats, mask, x_len, W, b, gamma, beta):
    total, feat_len = feats.shape
    n_feat = mask.shape[1]
    bag = W.shape[1]
    rows = _ROWS_PER_BLOCK
    num_blocks = total // rows

    w1 = W[:feat_len]
    w2 = W[feat_len:]
    b2 = b.reshape(1, bag)
    gamma2 = gamma.reshape(1, bag)
    beta2 = beta.reshape(1, bag)
    lens2 = jnp.broadcast_to(x_len.reshape(_BATCH, 1), (_BATCH, 128))

    grid = (num_blocks,)
    kern = functools.partial(_fused_kernel, rows_per_block=rows,
                             num_blocks=num_blocks)
    x_raw, x = pl.pallas_call(
        kern,
        grid=grid,
        in_specs=[
            pl.BlockSpec((_BATCH, 128), lambda i: (0, 0)),
            pl.BlockSpec((rows, feat_len), lambda i: (i, 0)),
            pl.BlockSpec((rows, n_feat), lambda i: (i, 0)),
            pl.BlockSpec((feat_len, bag), lambda i: (0, 0)),
            pl.BlockSpec((n_feat, bag), lambda i: (0, 0)),
            pl.BlockSpec((1, bag), lambda i: (0, 0)),
            pl.BlockSpec((1, bag), lambda i: (0, 0)),
            pl.BlockSpec((1, bag), lambda i: (0, 0)),
        ],
        out_specs=[
            pl.BlockSpec((rows, bag), lambda i: (i, 0)),
            pl.BlockSpec((_BATCH, bag), lambda i: (0, 0)),
        ],
        out_shape=[
            jax.ShapeDtypeStruct((total, bag), jnp.float32),
            jax.ShapeDtypeStruct((_BATCH, bag), jnp.float32),
        ],
        scratch_shapes=[pltpu.VMEM((_BATCH, bag), jnp.float32)],
        compiler_params=pltpu.CompilerParams(
            dimension_semantics=("arbitrary",),
        ),
    )(lens2, feats, mask, w1, w2, b2, gamma2, beta2)
    return (x, x_raw, mask)


# speedup vs baseline: 4.4928x; 1.0844x over previous
import functools

import jax
import jax.numpy as jnp
from jax.experimental import pallas as pl
from jax.experimental.pallas import tpu as pltpu

_BATCH = 16
_ROWS_PER_BLOCK = 2048


def _fused_kernel(lens_ref, feats_ref, mask_ref, w1_ref, w2_ref, b_ref,
                  gamma_ref, beta_ref, xraw_ref, x_ref, acc_ref,
                  *, rows_per_block, num_blocks):
    i = pl.program_id(0)

    y = jnp.dot(feats_ref[...], w1_ref[...], preferred_element_type=jnp.float32)
    y = y + jnp.dot(mask_ref[...], w2_ref[...], preferred_element_type=jnp.float32)
    y = y + b_ref[...]
    y = jnp.where(y >= 0.0, y, 0.01 * y)
    xraw_ref[...] = y

    lens_col = lens_ref[:, 0:1].astype(jnp.float32)
    r = jax.lax.broadcasted_iota(jnp.int32, (_BATCH, _BATCH), 0)
    c = jax.lax.broadcasted_iota(jnp.int32, (_BATCH, _BATCH), 1)
    tril = (c < r).astype(jnp.float32)
    starts = jnp.dot(tril, lens_col, preferred_element_type=jnp.float32,
                     precision=jax.lax.Precision.HIGHEST)
    ends = starts + lens_col

    row_idx = (i * rows_per_block
               + jax.lax.broadcasted_iota(jnp.int32, (_BATCH, rows_per_block), 1)
               ).astype(jnp.float32)
    onehot = ((row_idx >= starts) & (row_idx < ends)).astype(jnp.float32)
    partial = jnp.dot(onehot, y, preferred_element_type=jnp.float32,
                      precision=jax.lax.Precision.HIGHEST)

    @pl.when(i == 0)
    def _init():
        acc_ref[...] = partial

    @pl.when(i > 0)
    def _accum():
        acc_ref[...] = acc_ref[...] + partial

    @pl.when(i == num_blocks - 1)
    def _finalize():
        mean = acc_ref[...] / lens_col
        mu = jnp.mean(mean, axis=-1, keepdims=True)
        var = jnp.mean((mean - mu) ** 2, axis=-1, keepdims=True)
        x_ref[...] = ((mean - mu) / jnp.sqrt(var + 1e-5)
                      * gamma_ref[...] + beta_ref[...])


def kernel(feats, mask, x_len, W, b, gamma, beta):
    total, feat_len = feats.shape
    n_feat = mask.shape[1]
    bag = W.shape[1]
    rows = _ROWS_PER_BLOCK
    num_blocks = total // rows

    w1 = W[:feat_len]
    w2 = W[feat_len:]
    b2 = b.reshape(1, bag)
    gamma2 = gamma.reshape(1, bag)
    beta2 = beta.reshape(1, bag)
    lens2 = jnp.broadcast_to(x_len.reshape(_BATCH, 1), (_BATCH, 128))

    grid = (num_blocks,)
    kern = functools.partial(_fused_kernel, rows_per_block=rows,
                             num_blocks=num_blocks)
    x_raw, x = pl.pallas_call(
        kern,
        grid=grid,
        in_specs=[
            pl.BlockSpec((_BATCH, 128), lambda i: (0, 0)),
            pl.BlockSpec((rows, feat_len), lambda i: (i, 0)),
            pl.BlockSpec((rows, n_feat), lambda i: (i, 0)),
            pl.BlockSpec((feat_len, bag), lambda i: (0, 0)),
            pl.BlockSpec((n_feat, bag), lambda i: (0, 0)),
            pl.BlockSpec((1, bag), lambda i: (0, 0)),
            pl.BlockSpec((1, bag), lambda i: (0, 0)),
            pl.BlockSpec((1, bag), lambda i: (0, 0)),
        ],
        out_specs=[
            pl.BlockSpec((rows, bag), lambda i: (i, 0)),
            pl.BlockSpec((_BATCH, bag), lambda i: (0, 0)),
        ],
        out_shape=[
            jax.ShapeDtypeStruct((total, bag), jnp.float32),
            jax.ShapeDtypeStruct((_BATCH, bag), jnp.float32),
        ],
        scratch_shapes=[pltpu.VMEM((_BATCH, bag), jnp.float32)],
        compiler_params=pltpu.CompilerParams(
            dimension_semantics=("arbitrary",),
        ),
    )(lens2, feats, mask, w1, w2, b2, gamma2, beta2)
    return (x, x_raw, mask)
